# Initial kernel scaffold; baseline (speedup 1.0000x reference)
import jax
import jax.numpy as jnp
from jax.experimental import pallas as pl


def kernel(inputs, tables, w, b, W1, B1, g1, be1, W2, B2, g2, be2, W3, B3, g3, be3, Wo, Bo):
    raise NotImplementedError("write your pallas kernel here")


# speedup vs baseline: 2.9711x; 2.9711x over previous
import functools

import jax
import jax.numpy as jnp
from jax import lax
from jax.experimental import pallas as pl
from jax.experimental.pallas import tpu as pltpu
from jax.experimental.pallas import tpu_sc as plsc

B = 4096
N_DENSE = 13
N_SPARSE = 26
VOCAB = 1000
EDIM = 128
EPS = 1e-3
ROWS = B * N_SPARSE
NW = 32
RPW = ROWS // NW
CH = 128
NCHUNK = RPW // CH
NCHUNK_PAD = 32
NBUF = 2

@functools.cache
def _make_sc_gather():
    mesh = plsc.VectorSubcoreMesh(core_axis_name="c", subcore_axis_name="s")
    return pl.kernel(
        _sc_gather_body,
        out_type=jax.ShapeDtypeStruct((ROWS, EDIM), jnp.float32),
        mesh=mesh,
        scratch_types=[
            pltpu.VMEM((NCHUNK_PAD, CH), jnp.int32),
            pltpu.VMEM((CH, EDIM), jnp.float32),
            pltpu.VMEM((CH, EDIM), jnp.float32),
            pltpu.SemaphoreType.DMA,
            pltpu.SemaphoreType.DMA,
        ],
    )


def _sc_gather_body(table_hbm, idx_hbm, out_hbm, idx_v, buf0, buf1, sem0,
                    sem1):
    wid = lax.axis_index("s") * 2 + lax.axis_index("c")
    pltpu.sync_copy(idx_hbm.at[wid], idx_v)
    base = wid * RPW
    bufs = (buf0, buf1)
    sems = (sem0, sem1)
    for b in range(NBUF):
        pltpu.async_copy(table_hbm.at[idx_v.at[b]], bufs[b], sems[b])

    def outer(j0, carry):
        for b in range(NBUF):
            j = j0 * NBUF + b
            pltpu.make_async_copy(table_hbm.at[idx_v.at[b]], bufs[b],
                                  sems[b]).wait()
            pltpu.sync_copy(bufs[b], out_hbm.at[pl.ds(base + j * CH, CH)])

            @pl.when(j + NBUF < NCHUNK)
            def _():
                pltpu.async_copy(table_hbm.at[idx_v.at[j + NBUF]], bufs[b],
                                 sems[b])
        return carry

    lax.fori_loop(0, NCHUNK // NBUF, outer, 0)


_S = float(1.0 / (1.0 + EPS) ** 0.5)


def _mlp_body(inp_ref, emb_ref, w_ref, W1s_ref, W1d_ref, P1_ref, W2_ref,
              P2_ref, W3_ref, P3_ref, Wo_ref, wb_ref, out_ref):
    f32 = jnp.float32
    inp = inp_ref[...]
    h = jnp.dot(emb_ref[...], W1s_ref[...], preferred_element_type=f32)
    h = h + jnp.dot(inp, W1d_ref[...], preferred_element_type=f32)
    h = jnp.maximum((h + P1_ref[0:1, :]) * (P1_ref[1:2, :] * _S)
                    + P1_ref[2:3, :], 0.0)
    h = jnp.dot(h, W2_ref[...], preferred_element_type=f32)
    h = jnp.maximum((h + P2_ref[0:1, :]) * (P2_ref[1:2, :] * _S)
                    + P2_ref[2:3, :], 0.0)
    h = jnp.dot(h, W3_ref[...], preferred_element_type=f32)
    h = jnp.maximum((h + P3_ref[0:1, :]) * (P3_ref[1:2, :] * _S)
                    + P3_ref[2:3, :], 0.0)
    deep = jnp.dot(h, Wo_ref[...], preferred_element_type=f32)
    wide = jnp.dot(inp, w_ref[...], preferred_element_type=f32)
    out_ref[...] = jax.nn.sigmoid(deep + wide + wb_ref[...])


def _mlp_call(inputs_pad, emb, w_ext, W1s, W1d_ext, P1, W2, P2, W3, P3,
              Wo, wb):
    blk = 512
    grid = (B // blk,)
    full = lambda a: pl.BlockSpec(a.shape, lambda i: (0,) * a.ndim)
    in_specs = [
        pl.BlockSpec((blk, 128), lambda i: (i, 0)),
        pl.BlockSpec((blk, N_SPARSE * EDIM), lambda i: (i, 0)),
        full(w_ext), full(W1s), full(W1d_ext), full(P1), full(W2),
        full(P2), full(W3), full(P3), full(Wo), full(wb),
    ]
    return pl.pallas_call(
        _mlp_body,
        grid=grid,
        in_specs=in_specs,
        out_specs=pl.BlockSpec((blk, 1), lambda i: (i, 0)),
        out_shape=jax.ShapeDtypeStruct((B, 1), jnp.float32),
    )(inputs_pad, emb, w_ext, W1s, W1d_ext, P1, W2, P2, W3, P3, Wo, wb)


def kernel(inputs, tables, w, b, W1, B1, g1, be1, W2, B2, g2, be2, W3, B3,
           g3, be3, Wo, Bo):
    idx = inputs[:, N_DENSE:].astype(jnp.int32)
    flat_idx = (idx + jnp.arange(N_SPARSE, dtype=jnp.int32)[None, :] * VOCAB)
    idx3d = flat_idx.reshape(NW, NCHUNK, CH)
    idx3d = jnp.pad(idx3d, ((0, 0), (0, NCHUNK_PAD - NCHUNK), (0, 0)))
    tables_flat = tables.reshape(N_SPARSE * VOCAB, EDIM)

    emb = _make_sc_gather()(tables_flat, idx3d)

    inputs_pad = jnp.pad(inputs, ((0, 0), (0, 128 - (N_DENSE + N_SPARSE))))
    w_ext = jnp.pad(w, ((0, 128 - (N_DENSE + N_SPARSE)), (0, 0)))
    W1s = W1[: N_SPARSE * EDIM]
    W1d_ext = jnp.pad(W1[N_SPARSE * EDIM:], ((0, 128 - N_DENSE), (0, 0)))
    P1 = jnp.stack([B1, g1, be1])
    P2 = jnp.stack([B2, g2, be2])
    P3 = jnp.stack([B3, g3, be3])
    wb = (b + Bo).reshape(1, 1)

    return _mlp_call(inputs_pad, emb.reshape(B, N_SPARSE * EDIM), w_ext,
                     W1s, W1d_ext, P1, W2, P2, W3, P3, Wo, wb)


# speedup vs baseline: 3.0393x; 1.0230x over previous
import functools

import jax
import jax.numpy as jnp
from jax import lax
from jax.experimental import pallas as pl
from jax.experimental.pallas import tpu as pltpu
from jax.experimental.pallas import tpu_sc as plsc

B = 4096
N_DENSE = 13
N_SPARSE = 26
VOCAB = 1000
EDIM = 128
EPS = 1e-3
ROWS = B * N_SPARSE
NW = 32
RPW = ROWS // NW
CH = 128
NCHUNK = RPW // CH
NCHUNK_PAD = 32
NBUF = 2

@functools.cache
def _make_sc_gather():
    mesh = plsc.VectorSubcoreMesh(core_axis_name="c", subcore_axis_name="s")
    return pl.kernel(
        _sc_gather_body,
        out_type=jax.ShapeDtypeStruct((ROWS, EDIM), jnp.float32),
        mesh=mesh,
        scratch_types=[
            pltpu.VMEM((NCHUNK_PAD, CH), jnp.int32),
            pltpu.VMEM((CH, EDIM), jnp.float32),
            pltpu.VMEM((CH, EDIM), jnp.float32),
            pltpu.SemaphoreType.DMA,
            pltpu.SemaphoreType.DMA,
        ],
    )


def _sc_gather_body(table_hbm, idx_hbm, out_hbm, idx_v, buf0, buf1, sem0,
                    sem1):
    wid = lax.axis_index("s") * 2 + lax.axis_index("c")
    pltpu.sync_copy(idx_hbm.at[wid], idx_v)
    base = wid * RPW
    bufs = (buf0, buf1)
    sems = (sem0, sem1)
    for b in range(NBUF):
        pltpu.async_copy(table_hbm.at[idx_v.at[b]], bufs[b], sems[b])

    def outer(j0, carry):
        for b in range(NBUF):
            j = j0 * NBUF + b
            pltpu.make_async_copy(table_hbm.at[idx_v.at[b]], bufs[b],
                                  sems[b]).wait()
            pltpu.sync_copy(bufs[b], out_hbm.at[pl.ds(base + j * CH, CH)])

            @pl.when(j + NBUF < NCHUNK)
            def _():
                pltpu.async_copy(table_hbm.at[idx_v.at[j + NBUF]], bufs[b],
                                 sems[b])
        return carry

    lax.fori_loop(0, NCHUNK // NBUF, outer, 0)


_S = float(1.0 / (1.0 + EPS) ** 0.5)


def _mlp_body(inp_ref, emb_ref, w_ref, W1s_ref, W1d_ref, P1_ref, W2_ref,
              P2_ref, W3_ref, P3_ref, Wo_ref, wb_ref, out_ref):
    f32 = jnp.float32
    bf16 = jnp.bfloat16
    inp = inp_ref[...]
    h = jnp.dot(emb_ref[...].astype(bf16), W1s_ref[...],
                preferred_element_type=f32)
    h = h + jnp.dot(inp, W1d_ref[...], preferred_element_type=f32)
    h = jnp.maximum((h + P1_ref[0:1, :]) * (P1_ref[1:2, :] * _S)
                    + P1_ref[2:3, :], 0.0)
    h = jnp.dot(h.astype(bf16), W2_ref[...], preferred_element_type=f32)
    h = jnp.maximum((h + P2_ref[0:1, :]) * (P2_ref[1:2, :] * _S)
                    + P2_ref[2:3, :], 0.0)
    h = jnp.dot(h.astype(bf16), W3_ref[...], preferred_element_type=f32)
    h = jnp.maximum((h + P3_ref[0:1, :]) * (P3_ref[1:2, :] * _S)
                    + P3_ref[2:3, :], 0.0)
    deep = jnp.dot(h.astype(bf16), Wo_ref[...], preferred_element_type=f32)
    wide = jnp.dot(inp, w_ref[...], preferred_element_type=f32)
    out_ref[...] = jax.nn.sigmoid(deep + wide + wb_ref[...])


def _mlp_call(inputs_pad, emb, w_ext, W1s, W1d_ext, P1, W2, P2, W3, P3,
              Wo, wb):
    blk = 512
    grid = (B // blk,)
    full = lambda a: pl.BlockSpec(a.shape, lambda i: (0,) * a.ndim)
    in_specs = [
        pl.BlockSpec((blk, 128), lambda i: (i, 0)),
        pl.BlockSpec((blk, N_SPARSE * EDIM), lambda i: (i, 0)),
        full(w_ext), full(W1s), full(W1d_ext), full(P1), full(W2),
        full(P2), full(W3), full(P3), full(Wo), full(wb),
    ]
    return pl.pallas_call(
        _mlp_body,
        grid=grid,
        in_specs=in_specs,
        out_specs=pl.BlockSpec((blk, 1), lambda i: (i, 0)),
        out_shape=jax.ShapeDtypeStruct((B, 1), jnp.float32),
    )(inputs_pad, emb, w_ext, W1s, W1d_ext, P1, W2, P2, W3, P3, Wo, wb)


def kernel(inputs, tables, w, b, W1, B1, g1, be1, W2, B2, g2, be2, W3, B3,
           g3, be3, Wo, Bo):
    idx = inputs[:, N_DENSE:].astype(jnp.int32)
    flat_idx = (idx + jnp.arange(N_SPARSE, dtype=jnp.int32)[None, :] * VOCAB)
    idx3d = flat_idx.reshape(NW, NCHUNK, CH)
    idx3d = jnp.pad(idx3d, ((0, 0), (0, NCHUNK_PAD - NCHUNK), (0, 0)))
    tables_flat = tables.reshape(N_SPARSE * VOCAB, EDIM)

    emb = _make_sc_gather()(tables_flat, idx3d)

    inputs_pad = jnp.pad(inputs, ((0, 0), (0, 128 - (N_DENSE + N_SPARSE))))
    w_ext = jnp.pad(w, ((0, 128 - (N_DENSE + N_SPARSE)), (0, 0)))
    W1s = W1[: N_SPARSE * EDIM].astype(jnp.bfloat16)
    W1d_ext = jnp.pad(W1[N_SPARSE * EDIM:], ((0, 128 - N_DENSE), (0, 0)))
    P1 = jnp.stack([B1, g1, be1])
    P2 = jnp.stack([B2, g2, be2])
    P3 = jnp.stack([B3, g3, be3])
    wb = (b + Bo).reshape(1, 1)

    return _mlp_call(inputs_pad, emb.reshape(B, N_SPARSE * EDIM), w_ext,
                     W1s, W1d_ext, P1, W2.astype(jnp.bfloat16), P2,
                     W3.astype(jnp.bfloat16), P3, Wo.astype(jnp.bfloat16),
                     wb)


# speedup vs baseline: 4.4787x; 1.4736x over previous
import functools

import jax
import jax.numpy as jnp
from jax import lax
from jax.experimental import pallas as pl
from jax.experimental.pallas import tpu as pltpu
from jax.experimental.pallas import tpu_sc as plsc

B = 4096
N_DENSE = 13
N_SPARSE = 26
VOCAB = 1000
EDIM = 128
EPS = 1e-3
ROWS = B * N_SPARSE
NW = 32
RPW = ROWS // NW
CH = 128
NCHUNK = RPW // CH
NCHUNK_PAD = 32
NBUF = 2

@functools.cache
def _make_sc_gather():
    mesh = plsc.VectorSubcoreMesh(core_axis_name="c", subcore_axis_name="s")
    return pl.kernel(
        _sc_gather_body,
        out_type=jax.ShapeDtypeStruct((B, N_SPARSE * EDIM), jnp.float32),
        mesh=mesh,
        scratch_types=[
            pltpu.VMEM((NCHUNK_PAD, CH), jnp.int32),
            pltpu.VMEM((CH, EDIM), jnp.float32),
            pltpu.VMEM((CH, EDIM), jnp.float32),
            pltpu.SemaphoreType.DMA,
            pltpu.SemaphoreType.DMA,
        ],
    )


def _sc_gather_body(table_hbm, idx_hbm, out_hbm, idx_v, buf0, buf1, sem0,
                    sem1):
    wid = lax.axis_index("s") * 2 + lax.axis_index("c")
    pltpu.sync_copy(idx_hbm.at[wid], idx_v)
    bufs = (buf0, buf1)
    sems = (sem0, sem1)
    for b in range(NBUF):
        pltpu.async_copy(table_hbm.at[idx_v.at[b]], bufs[b], sems[b])

    def outer(j0, carry):
        for b in range(NBUF):
            j = j0 * NBUF + b
            c = wid * NCHUNK + j
            field = c // (B // CH)
            brow = (c % (B // CH)) * CH
            pltpu.make_async_copy(table_hbm.at[idx_v.at[b]], bufs[b],
                                  sems[b]).wait()
            pltpu.sync_copy(
                bufs[b],
                out_hbm.at[pl.ds(brow, CH), pl.ds(field * EDIM, EDIM)])

            @pl.when(j + NBUF < NCHUNK)
            def _():
                pltpu.async_copy(table_hbm.at[idx_v.at[j + NBUF]], bufs[b],
                                 sems[b])
        return carry

    lax.fori_loop(0, NCHUNK // NBUF, outer, 0)


_S = float(1.0 / (1.0 + EPS) ** 0.5)


def _mlp_body(inp_ref, emb_ref, w_ref, W1s_ref, W1d_ref, P1_ref, W2_ref,
              P2_ref, W3_ref, P3_ref, Wo_ref, wb_ref, out_ref):
    f32 = jnp.float32
    bf16 = jnp.bfloat16
    inp = inp_ref[...]
    h = jnp.dot(emb_ref[...].astype(bf16), W1s_ref[...],
                preferred_element_type=f32)
    h = h + jnp.dot(inp, W1d_ref[...], preferred_element_type=f32)
    h = jnp.maximum((h + P1_ref[0:1, :]) * (P1_ref[1:2, :] * _S)
                    + P1_ref[2:3, :], 0.0)
    h = jnp.dot(h.astype(bf16), W2_ref[...], preferred_element_type=f32)
    h = jnp.maximum((h + P2_ref[0:1, :]) * (P2_ref[1:2, :] * _S)
                    + P2_ref[2:3, :], 0.0)
    h = jnp.dot(h.astype(bf16), W3_ref[...], preferred_element_type=f32)
    h = jnp.maximum((h + P3_ref[0:1, :]) * (P3_ref[1:2, :] * _S)
                    + P3_ref[2:3, :], 0.0)
    deep = jnp.dot(h.astype(bf16), Wo_ref[...], preferred_element_type=f32)
    wide = jnp.dot(inp, w_ref[...], preferred_element_type=f32)
    out_ref[...] = jax.nn.sigmoid(deep + wide + wb_ref[...])


def _mlp_call(inputs_pad, emb, w_ext, W1s, W1d_ext, P1, W2, P2, W3, P3,
              Wo, wb):
    blk = 512
    grid = (B // blk,)
    full = lambda a: pl.BlockSpec(a.shape, lambda i: (0,) * a.ndim)
    in_specs = [
        pl.BlockSpec((blk, 128), lambda i: (i, 0)),
        pl.BlockSpec((blk, N_SPARSE * EDIM), lambda i: (i, 0)),
        full(w_ext), full(W1s), full(W1d_ext), full(P1), full(W2),
        full(P2), full(W3), full(P3), full(Wo), full(wb),
    ]
    return pl.pallas_call(
        _mlp_body,
        grid=grid,
        in_specs=in_specs,
        out_specs=pl.BlockSpec((blk, 1), lambda i: (i, 0)),
        out_shape=jax.ShapeDtypeStruct((B, 1), jnp.float32),
    )(inputs_pad, emb, w_ext, W1s, W1d_ext, P1, W2, P2, W3, P3, Wo, wb)


def kernel(inputs, tables, w, b, W1, B1, g1, be1, W2, B2, g2, be2, W3, B3,
           g3, be3, Wo, Bo):
    idx = inputs[:, N_DENSE:].astype(jnp.int32)
    flat_idx = (idx + jnp.arange(N_SPARSE, dtype=jnp.int32)[None, :] * VOCAB)
    idx3d = flat_idx.T.reshape(NW, NCHUNK, CH)
    idx3d = jnp.pad(idx3d, ((0, 0), (0, NCHUNK_PAD - NCHUNK), (0, 0)))
    tables_flat = tables.reshape(N_SPARSE * VOCAB, EDIM)

    emb = _make_sc_gather()(tables_flat, idx3d)

    inputs_pad = jnp.pad(inputs, ((0, 0), (0, 128 - (N_DENSE + N_SPARSE))))
    w_ext = jnp.pad(w, ((0, 128 - (N_DENSE + N_SPARSE)), (0, 0)))
    W1s = W1[: N_SPARSE * EDIM].astype(jnp.bfloat16)
    W1d_ext = jnp.pad(W1[N_SPARSE * EDIM:], ((0, 128 - N_DENSE), (0, 0)))
    P1 = jnp.stack([B1, g1, be1])
    P2 = jnp.stack([B2, g2, be2])
    P3 = jnp.stack([B3, g3, be3])
    wb = (b + Bo).reshape(1, 1)

    return _mlp_call(inputs_pad, emb, w_ext,
                     W1s, W1d_ext, P1, W2.astype(jnp.bfloat16), P2,
                     W3.astype(jnp.bfloat16), P3, Wo.astype(jnp.bfloat16),
                     wb)
